# baseline (speedup 1.0000x reference)
import jax
import jax.numpy as jnp
from jax.experimental import pallas as pl
from jax.experimental.pallas import tpu as pltpu

_B = 256
_DIN = 1024
_D = 2048
_C = 1000
_K = 100
_N = _C + _B
_NP = 1280
_CP = 1024
_RB = 200
_ZC = 512


def _row_block_stats(lhs, w, b):
    logits = jax.lax.dot_general(
        lhs, w, (((1,), (1,)), ((), ())),
        preferred_element_type=jnp.float32) + b
    m = jnp.max(logits, axis=1, keepdims=True)
    e = jnp.exp(logits - m)
    s = jnp.sum(e, axis=1, keepdims=True)
    t = jnp.sum(e * logits, axis=1, keepdims=True)
    cols = jax.lax.broadcasted_iota(jnp.int32, logits.shape, 1)
    bi = jnp.min(jnp.where(logits == m, cols, jnp.int32(2**30)),
                 axis=1, keepdims=True)
    ent = m + jnp.log(s) - t / s
    rn2 = jnp.sum(lhs * lhs, axis=1, keepdims=True)
    invn = jax.lax.rsqrt(jnp.maximum(rn2, 1e-24))
    return ent, bi.astype(jnp.float32), invn


def _body(w_ref, x_ref, fw_ref, fb_ref, b_ref, o_ref,
          z_sc, ent_sc, yf_sc, inv_sc):
    s = pl.program_id(0)

    @pl.when(s <= 3)
    def _z_chunk():
        zc = jax.lax.dot_general(
            x_ref[...], fw_ref[...], (((1,), (1,)), ((), ())),
            preferred_element_type=jnp.float32) + fb_ref[...]
        z_sc[:, pl.ds(s * _ZC, _ZC)] = zc

    @pl.when(s < 5)
    def _warmup_rows():
        lhs = w_ref[pl.ds(s * _RB, _RB), :]
        ent, yf, invn = _row_block_stats(lhs, w_ref[...], b_ref[...])
        ent_sc[pl.ds(s * _RB, _RB), :] = ent
        yf_sc[pl.ds(s * _RB, _RB), :] = yf
        inv_sc[pl.ds(s * _RB, _RB), :] = invn

    @pl.when(s == 5)
    def _batch_rows():
        ent, yf, invn = _row_block_stats(z_sc[...], w_ref[...], b_ref[...])
        ent_sc[pl.ds(_C, _B), :] = ent
        yf_sc[pl.ds(_C, _B), :] = yf
        inv_sc[pl.ds(_C, _B), :] = invn
        yf_sc[pl.ds(_N, _NP - _N), :] = jnp.full(
            (_NP - _N, 1), -1.0, jnp.float32)

    @pl.when(s == 6)
    def _filter_and_out():
        ent_c = ent_sc[...]
        yf_c = yf_sc[...]
        idx_r = jax.lax.broadcasted_iota(jnp.int32, (_NP, _NP), 1)
        idx_c = jax.lax.broadcasted_iota(jnp.int32, (_NP, _NP), 0)
        cat = jnp.concatenate([ent_c, yf_c], axis=1)
        rows = jnp.transpose(cat)
        ent_r = rows[0:1, :]
        yf_r = rows[1:2, :]
        same = yf_r == yf_c
        earlier = (ent_r < ent_c) | ((ent_r == ent_c) & (idx_r < idx_c))
        rank = jnp.sum(same & earlier, axis=1,
                       keepdims=True)
        valid = (idx_c[:, :1] < _N) & (rank < _K)
        coef = jnp.where(valid, inv_sc[...], 0.0)

        y_i32 = yf_c.astype(jnp.int32)
        yA = y_i32[:_C, :]
        cA = coef[:_C, :]
        yB = y_i32[_C:_N, :]
        cB = coef[_C:_N, :]
        clsA = jax.lax.broadcasted_iota(jnp.int32, (_C, _CP), 1)
        clsB = jax.lax.broadcasted_iota(jnp.int32, (_B, _CP), 1)
        ohA = jnp.where(yA == clsA, cA, 0.0).astype(jnp.bfloat16)
        ohB = jnp.where(yB == clsB, cB, 0.0).astype(jnp.bfloat16)
        z = z_sc[...]
        wT = jax.lax.dot_general(
            ohA, w_ref[...].astype(jnp.bfloat16), (((0,), (0,)), ((), ())),
            preferred_element_type=jnp.float32)
        wT = wT + jax.lax.dot_general(
            ohB, z.astype(jnp.bfloat16), (((0,), (0,)), ((), ())),
            preferred_element_type=jnp.float32)
        wn2 = jnp.sum(wT * wT, axis=1, keepdims=True)
        wn = wT * jax.lax.rsqrt(jnp.maximum(wn2, 1e-24))
        res = jax.lax.dot_general(
            z, wn, (((1,), (1,)), ((), ())),
            preferred_element_type=jnp.float32)
        o_ref[...] = res[:, :_C]


def kernel(x, feat_W, feat_b, cls_W, cls_b):
    return pl.pallas_call(
        _body,
        grid=(7,),
        in_specs=[
            pl.BlockSpec((_C, _D), lambda s: (0, 0)),
            pl.BlockSpec((_B, _DIN), lambda s: (0, 0)),
            pl.BlockSpec((_ZC, _DIN), lambda s: (jnp.minimum(s, 3), 0)),
            pl.BlockSpec((1, _ZC), lambda s: (0, jnp.minimum(s, 3))),
            pl.BlockSpec((1, _C), lambda s: (0, 0)),
        ],
        out_specs=pl.BlockSpec((_B, _C), lambda s: (0, 0)),
        out_shape=jax.ShapeDtypeStruct((_B, _C), jnp.float32),
        scratch_shapes=[
            pltpu.VMEM((_B, _D), jnp.float32),
            pltpu.VMEM((_NP, 1), jnp.float32),
            pltpu.VMEM((_NP, 1), jnp.float32),
            pltpu.VMEM((_NP, 1), jnp.float32),
        ],
        compiler_params=pltpu.CompilerParams(
            dimension_semantics=("arbitrary",)),
    )(cls_W, x, feat_W, feat_b.reshape(1, _D), cls_b.reshape(1, _C))
